# baseline (speedup 1.0000x reference)
import numpy as np
import jax
import jax.numpy as jnp
from jax.experimental import pallas as pl
from jax.experimental.pallas import tpu as pltpu
from jax.sharding import Mesh, PartitionSpec as P
from jax.experimental.shard_map import shard_map

BN = 1000


def _fused_heads(x_ref, wc_ref, bc_ref, wb_ref, bb_ref, sc_ref, bd_ref):
    x = x_ref[...]
    sc_ref[...] = (
        jnp.dot(x, wc_ref[...], preferred_element_type=jnp.float32) + bc_ref[...]
    )
    bd_ref[...] = (
        jnp.dot(x, wb_ref[...], preferred_element_type=jnp.float32) + bb_ref[...]
    )


def _pallas_fused(x, W_cls, bc, W_box, bb):
    n, d = x.shape
    kc = W_cls.shape[1]
    kb = W_box.shape[1]
    grid = (n // BN,)
    return pl.pallas_call(
        _fused_heads,
        grid=grid,
        in_specs=[
            pl.BlockSpec((BN, d), lambda i: (i, 0)),
            pl.BlockSpec((d, kc), lambda i: (0, 0)),
            pl.BlockSpec((1, kc), lambda i: (0, 0)),
            pl.BlockSpec((d, kb), lambda i: (0, 0)),
            pl.BlockSpec((1, kb), lambda i: (0, 0)),
        ],
        out_specs=[
            pl.BlockSpec((BN, kc), lambda i: (i, 0)),
            pl.BlockSpec((BN, kb), lambda i: (i, 0)),
        ],
        out_shape=[
            jax.ShapeDtypeStruct((n, kc), jnp.float32),
            jax.ShapeDtypeStruct((n, kb), jnp.float32),
        ],
    )(x, W_cls, bc, W_box, bb)


def kernel(x, W_cls, b_cls, W_box, b_box):
    kc = W_cls.shape[1]
    kb = W_box.shape[1]
    bc = b_cls.reshape(1, kc)
    bb = b_box.reshape(1, kb)
    devs = jax.devices()
    if len(devs) >= 2:
        mesh = Mesh(np.array(devs[:2]), ("r",))
        f = shard_map(
            _pallas_fused,
            mesh=mesh,
            in_specs=(P("r", None), P(None, None), P(None, None),
                      P(None, None), P(None, None)),
            out_specs=(P("r", None), P("r", None)),
            check_rep=False,
        )
        return tuple(f(x, W_cls, bc, W_box, bb))
    return tuple(_pallas_fused(x, W_cls, bc, W_box, bb))


# speedup vs baseline: 11.2568x; 11.2568x over previous
import jax
import jax.numpy as jnp
from jax import lax
from jax.experimental import pallas as pl

BN = 2048


def _fused_heads_t(x_ref, wct_ref, bct_ref, wbt_ref, bbt_ref, sct_ref, bdt_ref):
    x = x_ref[...]
    dims = (((1,), (1,)), ((), ()))
    sct_ref[...] = (
        lax.dot_general(wct_ref[...], x, dims, preferred_element_type=jnp.float32)
        + bct_ref[...]
    )
    bdt_ref[...] = (
        lax.dot_general(wbt_ref[...], x, dims, preferred_element_type=jnp.float32)
        + bbt_ref[...]
    )


def kernel(x, W_cls, b_cls, W_box, b_box):
    n, d = x.shape
    kc = W_cls.shape[1]
    kb = W_box.shape[1]
    wct = W_cls.T
    wbt = W_box.T
    bct = b_cls.reshape(kc, 1)
    bbt = b_box.reshape(kb, 1)
    grid = (pl.cdiv(n, BN),)
    sct, bdt = pl.pallas_call(
        _fused_heads_t,
        grid=grid,
        in_specs=[
            pl.BlockSpec((BN, d), lambda i: (i, 0)),
            pl.BlockSpec((kc, d), lambda i: (0, 0)),
            pl.BlockSpec((kc, 1), lambda i: (0, 0)),
            pl.BlockSpec((kb, d), lambda i: (0, 0)),
            pl.BlockSpec((kb, 1), lambda i: (0, 0)),
        ],
        out_specs=[
            pl.BlockSpec((kc, BN), lambda i: (0, i)),
            pl.BlockSpec((kb, BN), lambda i: (0, i)),
        ],
        out_shape=[
            jax.ShapeDtypeStruct((kc, n), jnp.float32),
            jax.ShapeDtypeStruct((kb, n), jnp.float32),
        ],
    )(x, wct, bct, wbt, bbt)
    return (sct.T, bdt.T)
